# baseline (speedup 1.0000x reference)
import dataclasses
import functools

import jax
import jax.numpy as jnp
from jax import lax
from jax.experimental import pallas as pl
from jax.experimental.pallas import tpu as pltpu
from jax.experimental.pallas import tpu_sc as plsc

NC, NS, L = 2, 16, 16
NW = NC * NS
CM = 64
CVW = 128


def _conv_body(x_ref, w_ref, b_ref, o_ref):
    x = x_ref[...].astype(jnp.bfloat16)
    acc = jnp.dot(x, w_ref[...], preferred_element_type=jnp.float32)
    o_ref[...] = jnp.maximum(acc + b_ref[...], 0.0)


def _conv_matmul(x_mod, w, b2d):
    P, D = x_mod.shape
    BM = 512
    return pl.pallas_call(
        _conv_body,
        grid=(P // BM,),
        in_specs=[
            pl.BlockSpec((BM, D), lambda i: (i, 0)),
            pl.BlockSpec((D, D), lambda i: (0, 0)),
            pl.BlockSpec((1, D), lambda i: (0, 0)),
        ],
        out_specs=pl.BlockSpec((BM, D), lambda i: (i, 0)),
        out_shape=jax.ShapeDtypeStruct((P, D), jnp.float32),
    )(x_mod, w, b2d)


def _score_body(xp_ref, w_ref, e_ref):
    x = xp_ref[...].astype(jnp.bfloat16)
    s = jnp.dot(x, w_ref[...], preferred_element_type=jnp.float32)
    e_ref[...] = jnp.exp(s[:, :1])


def _score_kernel(x_proj, wpad):
    A, DP = x_proj.shape
    BA = 4096
    return pl.pallas_call(
        _score_body,
        grid=(A // BA,),
        in_specs=[
            pl.BlockSpec((BA, DP), lambda i: (i, 0)),
            pl.BlockSpec((DP, 128), lambda i: (0, 0)),
        ],
        out_specs=pl.BlockSpec((BA, 1), lambda i: (i, 0)),
        out_shape=jax.ShapeDtypeStruct((A, 1), jnp.float32),
    )(x_proj, wpad)


def _part_body(x_ref, w_ref, b_ref, o_ref):
    x = x_ref[...].astype(jnp.bfloat16)
    acc = jnp.dot(x, w_ref[...], preferred_element_type=jnp.float32)
    o_ref[...] = acc + b_ref[...]


def _part_matmul(x_3d, w1, b2d):
    N, D = x_3d.shape
    BM = 512
    return pl.pallas_call(
        _part_body,
        grid=(N // BM,),
        in_specs=[
            pl.BlockSpec((BM, D), lambda i: (i, 0)),
            pl.BlockSpec((D, D), lambda i: (0, 0)),
            pl.BlockSpec((1, D), lambda i: (0, 0)),
        ],
        out_specs=pl.BlockSpec((BM, D), lambda i: (i, 0)),
        out_shape=jax.ShapeDtypeStruct((N, D), jnp.float32),
    )(x_3d, w1, b2d)


def _fuse_body(p_ref, v_ref, w2_ref, o_ref):
    v = v_ref[...].astype(jnp.bfloat16)
    acc = p_ref[...] + jnp.dot(v, w2_ref[...],
                               preferred_element_type=jnp.float32)
    o_ref[...] = jnp.maximum(acc, 0.0)


def _fuse_matmul(part, v, w2):
    N, D = part.shape
    BM = 512
    return pl.pallas_call(
        _fuse_body,
        grid=(N // BM,),
        in_specs=[
            pl.BlockSpec((BM, D), lambda i: (i, 0)),
            pl.BlockSpec((BM, D), lambda i: (i, 0)),
            pl.BlockSpec((D, D), lambda i: (0, 0)),
        ],
        out_specs=pl.BlockSpec((BM, D), lambda i: (i, 0)),
        out_shape=jax.ShapeDtypeStruct((N, D), jnp.float32),
    )(part, v, w2)


def _sload(ref, idx):
    return ref[pl.ds(idx, L)][0]


def _sc_body(PW, D, h_hbm, fmi_hbm, aseg_hbm, vseg_hbm, e_hbm,
             bnd_hbm, v_hbm, cnt_hbm,
             bnd_v, idx_big, aseg_big, ga_v, gb_v,
             vs_win, e_win, denom_v, cnt_v, pacc_v, stage_v, zero_v,
             sema, semb, semz):
    ND = D // L
    L2 = 2 * L
    ND2 = D // L2
    wid = lax.axis_index("s") * NC + lax.axis_index("c")
    pltpu.sync_copy(bnd_hbm, bnd_v)
    a0 = _sload(bnd_v, wid)
    a1 = _sload(bnd_v, wid + 1)
    m0 = _sload(bnd_v, 64 + wid)
    m1 = _sload(bnd_v, 64 + wid + 1)
    p0 = wid * PW

    zeros = jnp.zeros((L,), jnp.float32)
    zeros_b = jnp.zeros((L2,), jnp.bfloat16)

    @pl.loop(0, PW // L)
    def _(i):
        denom_v[pl.ds(i * L, L)] = zeros
        cnt_v[pl.ds(i * L, L)] = zeros

    @pl.loop(0, 16)
    def _(i):
        @pl.loop(0, ND)
        def _(d):
            zero_v[i, pl.ds(d * L, L)] = zeros
            stage_v[i, pl.ds(d * L, L)] = zeros

    @pl.loop(0, ND)
    def _(i):
        pacc_v[pl.ds(i * L, L)] = zeros

    @pl.loop(0, PW // 16)
    def _(i):
        off = pl.multiple_of(p0 + i * 16, 16)
        pltpu.async_copy(zero_v, v_hbm.at[pl.ds(off, 16)], semz)

    iota = lax.iota(jnp.int32, L)
    ones = jnp.ones((L,), jnp.float32)
    c0 = a0 // CVW
    c1 = jnp.maximum(c0, (a1 + CVW - 1) // CVW)

    def devchunk(c, _):
        base = pl.multiple_of(c * CVW, CVW)
        pltpu.sync_copy(vseg_hbm.at[pl.ds(base, CVW)],
                        vs_win.at[pl.ds(0, CVW)])
        pltpu.sync_copy(e_hbm.at[pl.ds(base, CVW)], e_win.at[pl.ds(0, CVW)])
        for k in range(CVW // L):
            g = base + k * L + iota
            act = (g >= a0) & (g < a1)
            idx = vs_win[pl.ds(k * L, L)] - p0
            idx = jnp.where(act, idx, 0)
            plsc.addupdate_scatter(denom_v, [idx], e_win[pl.ds(k * L, L)],
                                   mask=act)
            plsc.addupdate_scatter(cnt_v, [idx], ones, mask=act)
        return 0

    lax.fori_loop(c0, c1, devchunk, 0)
    pltpu.sync_copy(cnt_v, cnt_hbm.at[pl.ds(pl.multiple_of(p0, PW), PW)])

    @pl.loop(0, PW // 16)
    def _(i):
        off = pl.multiple_of(p0 + i * 16, 16)
        pltpu.make_async_copy(zero_v, v_hbm.at[pl.ds(off, 16)], semz).wait()

    def flush(blk):
        pltpu.sync_copy(stage_v, v_hbm.at[pl.ds(pl.multiple_of(blk, 16), 16)])
        @pl.loop(0, 16)
        def _(i):
            @pl.loop(0, ND)
            def _(d):
                stage_v[i, pl.ds(d * L, L)] = zeros

    def close_point(cur_p, blk):
        srow = cur_p - blk
        for d in range(ND):
            sl = pl.ds(d * L, L)
            stage_v[srow, sl] = pacc_v[sl]
            pacc_v[sl] = zeros

    BB = 512
    NSUB = BB // CM
    bb0 = m0 // BB
    bb1 = jnp.maximum(bb0, (m1 + BB - 1) // BB)

    def issue(sub, g_ref, sem):
        pltpu.async_copy(h_hbm.at[idx_big.at[pl.ds(sub * CM, CM)]],
                         g_ref, sem)

    def wait_gather(sub, g_ref, sem):
        pltpu.make_async_copy(h_hbm.at[idx_big.at[pl.ds(sub * CM, CM)]],
                              g_ref, sem).wait()

    def walk_chunk(base, soff, g_ref, carry):
        r_lo = jnp.maximum(m0 - base, 0)
        r_hi = jnp.maximum(r_lo, jnp.minimum(CM, m1 - base))

        def row_body(r, carry):
            scal, acc = carry
            cur_view = scal[0]
            aseg = _sload(aseg_big, soff + r)

            def same(carry):
                scal, acc = carry
                nacc = tuple(
                    jnp.maximum(acc[d], g_ref[r, pl.ds(d * L, L)])
                    for d in range(ND))
                return (scal, nacc)

            def diff(carry):
                scal, acc = carry
                cur_view, cur_e, cur_p, blk, wb = scal

                @pl.when(cur_view >= 0)
                def _():
                    for d in range(ND):
                        sl = pl.ds(d * L, L)
                        pacc_v[sl] = pacc_v[sl] + cur_e * acc[d]

                def refill(w):
                    nwb = pl.multiple_of((aseg // CVW) * CVW, CVW)
                    pltpu.sync_copy(e_hbm.at[pl.ds(nwb, CVW)],
                                    e_win.at[pl.ds(0, CVW)])
                    pltpu.sync_copy(vseg_hbm.at[pl.ds(nwb, CVW)],
                                    vs_win.at[pl.ds(0, CVW)])
                    return nwb

                wb = lax.cond(aseg - wb >= CVW, refill, lambda w: w, wb)
                off = aseg - wb
                pnew = _sload(vs_win, off)
                wvec = e_win[pl.ds(off, L)] / denom_v[pl.ds(pnew - p0, L)]
                enew = wvec[0]

                def new_point(blk):
                    @pl.when(cur_p >= 0)
                    def _():
                        close_point(cur_p, blk)
                    nblk = (pnew // 16) * 16

                    @pl.when((blk >= 0) & (nblk != blk))
                    def _():
                        flush(blk)
                    return nblk

                blk = lax.cond(pnew != cur_p, new_point, lambda b: b, blk)
                cur_p = jnp.where(pnew != cur_p, pnew, cur_p)

                nacc = tuple(g_ref[r, pl.ds(d * L, L)] for d in range(ND))
                return ((aseg, enew, cur_p, blk, wb), nacc)

            return lax.cond(aseg == cur_view, same, diff, carry)

        return lax.fori_loop(r_lo, r_hi, row_body, carry)

    init = ((jnp.int32(-1), jnp.float32(0.0), jnp.int32(-1), jnp.int32(-1),
             jnp.int32(-2 * CVW)),
            tuple(zeros for _ in range(ND)))

    def block_body(bb, carry):
        bbase = pl.multiple_of(bb * BB, BB)
        pltpu.sync_copy(fmi_hbm.at[pl.ds(bbase, BB)], idx_big)
        pltpu.sync_copy(aseg_hbm.at[pl.ds(bbase, BB)],
                        aseg_big.at[pl.ds(0, BB)])
        issue(0, ga_v, sema)
        for sub in range(NSUB):
            g_ref, sem = (ga_v, sema) if sub % 2 == 0 else (gb_v, semb)
            if sub + 1 < NSUB:
                ng, ns = (gb_v, semb) if sub % 2 == 0 else (ga_v, sema)
                issue(sub + 1, ng, ns)
            wait_gather(sub, g_ref, sem)
            carry = walk_chunk(bb * BB + sub * CM, sub * CM, g_ref, carry)
        return carry

    scal, acc = lax.fori_loop(bb0, bb1, block_body, init)
    cur_view, cur_e, cur_p, blk, _ = scal

    @pl.when(cur_view >= 0)
    def _():
        for d in range(ND):
            sl = pl.ds(d * L, L)
            pacc_v[sl] = pacc_v[sl] + cur_e * acc[d]

    @pl.when(cur_p >= 0)
    def _():
        close_point(cur_p, blk)
        flush(blk)


def _sc_pass(h, fmi, atomic_seg, view_seg, e_arr, bnd, N, PW):
    D = h.shape[1]
    mesh = plsc.VectorSubcoreMesh(core_axis_name="c", subcore_axis_name="s")
    f32 = jnp.float32
    cp = pltpu.CompilerParams()
    if "needs_layout_passes" in pltpu.CompilerParams.__dataclass_fields__:
        cp = dataclasses.replace(cp, needs_layout_passes=False)
    kern = pl.kernel(
        functools.partial(_sc_body, PW, D),
        out_type=[
            jax.ShapeDtypeStruct((N, D), f32),
            jax.ShapeDtypeStruct((N,), f32),
        ],
        mesh=mesh,
        compiler_params=cp,
        scratch_types=[
            pltpu.VMEM((128,), jnp.int32),
            pltpu.VMEM((512,), jnp.int32),
            pltpu.VMEM((512 + L,), jnp.int32),
            pltpu.VMEM((CM, D), f32),
            pltpu.VMEM((CM, D), f32),
            pltpu.VMEM((CVW + L,), jnp.int32),
            pltpu.VMEM((CVW + L,), f32),
            pltpu.VMEM((PW + L,), f32),
            pltpu.VMEM((PW,), f32),
            pltpu.VMEM((D,), f32),
            pltpu.VMEM((16, D), f32),
            pltpu.VMEM((16, D), f32),
            pltpu.SemaphoreType.DMA,
            pltpu.SemaphoreType.DMA,
            pltpu.SemaphoreType.DMA,
        ],
    )
    return kern(h, fmi, atomic_seg, view_seg, e_arr, bnd)


def kernel(x_3d, x_mod, x_proj, feature_map_indexing, atomic_seg, view_seg,
           W_conv, b_conv, w_score, W_fuse, b_fuse):
    N, D = x_3d.shape
    A = x_proj.shape[0]

    DP = x_proj.shape[1]
    wc_bf = W_conv.astype(jnp.bfloat16)
    w1_bf = W_fuse[:D].astype(jnp.bfloat16)
    w2_bf = W_fuse[D:].astype(jnp.bfloat16)
    wpad = jnp.zeros((DP, 128), jnp.float32).at[:, 0].set(w_score)
    wpad_bf = wpad.astype(jnp.bfloat16)

    h = _conv_matmul(x_mod, wc_bf, b_conv.reshape(1, D))
    e2d = _score_kernel(x_proj, wpad_bf)
    e_arr = e2d.reshape(A)
    part = _part_matmul(x_3d, w1_bf, b_fuse.reshape(1, D))

    PW = N // NW
    p_starts = (jnp.arange(NW + 1, dtype=jnp.int32) * PW)
    a_starts = jnp.searchsorted(view_seg, p_starts).astype(jnp.int32)
    m_starts = jnp.searchsorted(atomic_seg, a_starts).astype(jnp.int32)
    bnd = jnp.zeros((128,), jnp.int32)
    bnd = bnd.at[0:NW + 1].set(a_starts).at[64:64 + NW + 1].set(m_starts)

    v, counts = _sc_pass(h, feature_map_indexing.astype(jnp.int32),
                         atomic_seg.astype(jnp.int32),
                         view_seg.astype(jnp.int32), e_arr, bnd, N, PW)

    fused = _fuse_matmul(part, v, w2_bf)
    return fused, counts > 0


# speedup vs baseline: 1.0270x; 1.0270x over previous
import dataclasses
import functools

import jax
import jax.numpy as jnp
from jax import lax
from jax.experimental import pallas as pl
from jax.experimental.pallas import tpu as pltpu
from jax.experimental.pallas import tpu_sc as plsc

NC, NS, L = 2, 16, 16
NW = NC * NS
CM = 64
CVW = 128


def _conv_body(x_ref, w_ref, b_ref, o_ref):
    x = x_ref[...].astype(jnp.bfloat16)
    acc = jnp.dot(x, w_ref[...], preferred_element_type=jnp.float32)
    o_ref[...] = jnp.maximum(acc + b_ref[...], 0.0)


def _conv_matmul(x_mod, w, b2d):
    P, D = x_mod.shape
    BM = 512
    return pl.pallas_call(
        _conv_body,
        grid=(P // BM,),
        in_specs=[
            pl.BlockSpec((BM, D), lambda i: (i, 0)),
            pl.BlockSpec((D, D), lambda i: (0, 0)),
            pl.BlockSpec((1, D), lambda i: (0, 0)),
        ],
        out_specs=pl.BlockSpec((BM, D), lambda i: (i, 0)),
        out_shape=jax.ShapeDtypeStruct((P, D), jnp.float32),
    )(x_mod, w, b2d)


def _score_body(xp_ref, w_ref, e_ref):
    x = xp_ref[...].astype(jnp.bfloat16)
    s = jnp.dot(x, w_ref[...], preferred_element_type=jnp.float32)
    e_ref[...] = jnp.exp(s[:, :1])


def _score_kernel(x_proj, wpad):
    A, DP = x_proj.shape
    BA = 4096
    return pl.pallas_call(
        _score_body,
        grid=(A // BA,),
        in_specs=[
            pl.BlockSpec((BA, DP), lambda i: (i, 0)),
            pl.BlockSpec((DP, 128), lambda i: (0, 0)),
        ],
        out_specs=pl.BlockSpec((BA, 1), lambda i: (i, 0)),
        out_shape=jax.ShapeDtypeStruct((A, 1), jnp.float32),
    )(x_proj, wpad)


def _part_body(x_ref, w_ref, b_ref, o_ref):
    x = x_ref[...].astype(jnp.bfloat16)
    acc = jnp.dot(x, w_ref[...], preferred_element_type=jnp.float32)
    o_ref[...] = acc + b_ref[...]


def _part_matmul(x_3d, w1, b2d):
    N, D = x_3d.shape
    BM = 512
    return pl.pallas_call(
        _part_body,
        grid=(N // BM,),
        in_specs=[
            pl.BlockSpec((BM, D), lambda i: (i, 0)),
            pl.BlockSpec((D, D), lambda i: (0, 0)),
            pl.BlockSpec((1, D), lambda i: (0, 0)),
        ],
        out_specs=pl.BlockSpec((BM, D), lambda i: (i, 0)),
        out_shape=jax.ShapeDtypeStruct((N, D), jnp.float32),
    )(x_3d, w1, b2d)


def _fuse_body(p_ref, v_ref, w2_ref, o_ref):
    v = v_ref[...].astype(jnp.bfloat16)
    acc = p_ref[...] + jnp.dot(v, w2_ref[...],
                               preferred_element_type=jnp.float32)
    o_ref[...] = jnp.maximum(acc, 0.0)


def _fuse_matmul(part, v, w2):
    N, D = part.shape
    BM = 512
    return pl.pallas_call(
        _fuse_body,
        grid=(N // BM,),
        in_specs=[
            pl.BlockSpec((BM, D), lambda i: (i, 0)),
            pl.BlockSpec((BM, D), lambda i: (i, 0)),
            pl.BlockSpec((D, D), lambda i: (0, 0)),
        ],
        out_specs=pl.BlockSpec((BM, D), lambda i: (i, 0)),
        out_shape=jax.ShapeDtypeStruct((N, D), jnp.float32),
    )(part, v, w2)


def _sload(ref, idx):
    return ref[pl.ds(idx, L)][0]


def _sc_body(PW, D, h_hbm, fmi_hbm, aseg_hbm, vseg_hbm, e_hbm,
             bnd_hbm, v_hbm, cnt_hbm,
             bnd_v, idxa_v, idxb_v, asega_v, asegb_v, ga_v, gb_v,
             vs_win, e_win, denom_v, cnt_v, pacc_v, stage_v, zero_v,
             sema, semb, semz):
    ND = D // L
    L2 = 2 * L
    ND2 = D // L2
    wid = lax.axis_index("s") * NC + lax.axis_index("c")
    pltpu.sync_copy(bnd_hbm, bnd_v)
    a0 = _sload(bnd_v, wid)
    a1 = _sload(bnd_v, wid + 1)
    m0 = _sload(bnd_v, 64 + wid)
    m1 = _sload(bnd_v, 64 + wid + 1)
    p0 = wid * PW

    zeros = jnp.zeros((L,), jnp.float32)
    zeros_b = jnp.zeros((L2,), jnp.bfloat16)

    @pl.loop(0, PW // L)
    def _(i):
        denom_v[pl.ds(i * L, L)] = zeros
        cnt_v[pl.ds(i * L, L)] = zeros

    @pl.loop(0, 16)
    def _(i):
        @pl.loop(0, ND)
        def _(d):
            zero_v[i, pl.ds(d * L, L)] = zeros
            stage_v[i, pl.ds(d * L, L)] = zeros

    @pl.loop(0, ND)
    def _(i):
        pacc_v[pl.ds(i * L, L)] = zeros

    @pl.loop(0, PW // 16)
    def _(i):
        off = pl.multiple_of(p0 + i * 16, 16)
        pltpu.async_copy(zero_v, v_hbm.at[pl.ds(off, 16)], semz)

    iota = lax.iota(jnp.int32, L)
    ones = jnp.ones((L,), jnp.float32)
    c0 = a0 // CVW
    c1 = jnp.maximum(c0, (a1 + CVW - 1) // CVW)

    def devchunk(c, _):
        base = pl.multiple_of(c * CVW, CVW)
        pltpu.sync_copy(vseg_hbm.at[pl.ds(base, CVW)],
                        vs_win.at[pl.ds(0, CVW)])
        pltpu.sync_copy(e_hbm.at[pl.ds(base, CVW)], e_win.at[pl.ds(0, CVW)])
        for k in range(CVW // L):
            g = base + k * L + iota
            act = (g >= a0) & (g < a1)
            idx = vs_win[pl.ds(k * L, L)] - p0
            idx = jnp.where(act, idx, 0)
            plsc.addupdate_scatter(denom_v, [idx], e_win[pl.ds(k * L, L)],
                                   mask=act)
            plsc.addupdate_scatter(cnt_v, [idx], ones, mask=act)
        return 0

    lax.fori_loop(c0, c1, devchunk, 0)
    pltpu.sync_copy(cnt_v, cnt_hbm.at[pl.ds(pl.multiple_of(p0, PW), PW)])

    @pl.loop(0, PW // 16)
    def _(i):
        off = pl.multiple_of(p0 + i * 16, 16)
        pltpu.make_async_copy(zero_v, v_hbm.at[pl.ds(off, 16)], semz).wait()

    def flush(blk):
        pltpu.sync_copy(stage_v, v_hbm.at[pl.ds(pl.multiple_of(blk, 16), 16)])
        @pl.loop(0, 16)
        def _(i):
            @pl.loop(0, ND)
            def _(d):
                stage_v[i, pl.ds(d * L, L)] = zeros

    def close_point(cur_p, blk):
        srow = cur_p - blk
        for d in range(ND):
            sl = pl.ds(d * L, L)
            stage_v[srow, sl] = pacc_v[sl]
            pacc_v[sl] = zeros

    mc0 = m0 // CM
    mc1 = jnp.maximum(mc0, (m1 + CM - 1) // CM)
    nchunks = mc1 - mc0

    def issue(ci, idx_ref, aseg_ref, g_ref, sem):
        base = pl.multiple_of(ci * CM, CM)
        pltpu.sync_copy(fmi_hbm.at[pl.ds(base, CM)], idx_ref)
        pltpu.sync_copy(aseg_hbm.at[pl.ds(base, CM)],
                        aseg_ref.at[pl.ds(0, CM)])
        pltpu.async_copy(h_hbm.at[idx_ref], g_ref, sem)

    def wait_gather(idx_ref, g_ref, sem):
        pltpu.make_async_copy(h_hbm.at[idx_ref], g_ref, sem).wait()

    def walk_chunk(ci, g_ref, aseg_ref, carry):
        base = ci * CM
        r_lo = jnp.maximum(m0 - base, 0)
        r_hi = jnp.maximum(r_lo, jnp.minimum(CM, m1 - base))

        def row_body(r, carry):
            scal, acc = carry
            cur_view = scal[0]
            aseg = _sload(aseg_ref, r)

            def same(carry):
                scal, acc = carry
                nacc = tuple(
                    jnp.maximum(acc[d], g_ref[r, pl.ds(d * L, L)])
                    for d in range(ND))
                return (scal, nacc)

            def diff(carry):
                scal, acc = carry
                cur_view, cur_e, cur_p, blk, wb = scal

                @pl.when(cur_view >= 0)
                def _():
                    for d in range(ND):
                        sl = pl.ds(d * L, L)
                        pacc_v[sl] = pacc_v[sl] + cur_e * acc[d]

                def refill(w):
                    nwb = pl.multiple_of((aseg // CVW) * CVW, CVW)
                    pltpu.sync_copy(e_hbm.at[pl.ds(nwb, CVW)],
                                    e_win.at[pl.ds(0, CVW)])
                    pltpu.sync_copy(vseg_hbm.at[pl.ds(nwb, CVW)],
                                    vs_win.at[pl.ds(0, CVW)])
                    return nwb

                wb = lax.cond(aseg - wb >= CVW, refill, lambda w: w, wb)
                off = aseg - wb
                pnew = _sload(vs_win, off)
                wvec = e_win[pl.ds(off, L)] / denom_v[pl.ds(pnew - p0, L)]
                enew = wvec[0]

                def new_point(blk):
                    @pl.when(cur_p >= 0)
                    def _():
                        close_point(cur_p, blk)
                    nblk = (pnew // 16) * 16

                    @pl.when((blk >= 0) & (nblk != blk))
                    def _():
                        flush(blk)
                    return nblk

                blk = lax.cond(pnew != cur_p, new_point, lambda b: b, blk)
                cur_p = jnp.where(pnew != cur_p, pnew, cur_p)

                nacc = tuple(g_ref[r, pl.ds(d * L, L)] for d in range(ND))
                return ((aseg, enew, cur_p, blk, wb), nacc)

            return lax.cond(aseg == cur_view, same, diff, carry)

        return lax.fori_loop(r_lo, r_hi, row_body, carry)

    @pl.when(nchunks > 0)
    def _():
        issue(mc0, idxa_v, asega_v, ga_v, sema)

    init = ((jnp.int32(-1), jnp.float32(0.0), jnp.int32(-1), jnp.int32(-1),
             jnp.int32(-2 * CVW)),
            tuple(zeros for _ in range(ND)))
    npairs = (nchunks + 1) // 2

    def pair_body(k, carry):
        ci0 = mc0 + 2 * k
        ci1 = ci0 + 1

        @pl.when(ci1 < mc1)
        def _():
            issue(ci1, idxb_v, asegb_v, gb_v, semb)

        wait_gather(idxa_v, ga_v, sema)
        carry = walk_chunk(ci0, ga_v, asega_v, carry)

        @pl.when(ci0 + 2 < mc1)
        def _():
            issue(ci0 + 2, idxa_v, asega_v, ga_v, sema)

        def do_b(carry):
            wait_gather(idxb_v, gb_v, semb)
            return walk_chunk(ci1, gb_v, asegb_v, carry)

        return lax.cond(ci1 < mc1, do_b, lambda c: c, carry)

    scal, acc = lax.fori_loop(0, npairs, pair_body, init)
    cur_view, cur_e, cur_p, blk, _ = scal

    @pl.when(cur_view >= 0)
    def _():
        for d in range(ND):
            sl = pl.ds(d * L, L)
            pacc_v[sl] = pacc_v[sl] + cur_e * acc[d]

    @pl.when(cur_p >= 0)
    def _():
        close_point(cur_p, blk)
        flush(blk)


def _sc_pass(h, fmi, atomic_seg, view_seg, e_arr, bnd, N, PW):
    D = h.shape[1]
    mesh = plsc.VectorSubcoreMesh(core_axis_name="c", subcore_axis_name="s")
    f32 = jnp.float32
    cp = pltpu.CompilerParams()
    if "needs_layout_passes" in pltpu.CompilerParams.__dataclass_fields__:
        cp = dataclasses.replace(cp, needs_layout_passes=False)
    kern = pl.kernel(
        functools.partial(_sc_body, PW, D),
        out_type=[
            jax.ShapeDtypeStruct((N, D), f32),
            jax.ShapeDtypeStruct((N,), f32),
        ],
        mesh=mesh,
        compiler_params=cp,
        scratch_types=[
            pltpu.VMEM((128,), jnp.int32),
            pltpu.VMEM((CM,), jnp.int32),
            pltpu.VMEM((CM,), jnp.int32),
            pltpu.VMEM((CM + L,), jnp.int32),
            pltpu.VMEM((CM + L,), jnp.int32),
            pltpu.VMEM((CM, D), f32),
            pltpu.VMEM((CM, D), f32),
            pltpu.VMEM((CVW + L,), jnp.int32),
            pltpu.VMEM((CVW + L,), f32),
            pltpu.VMEM((PW + L,), f32),
            pltpu.VMEM((PW,), f32),
            pltpu.VMEM((D,), f32),
            pltpu.VMEM((16, D), f32),
            pltpu.VMEM((16, D), f32),
            pltpu.SemaphoreType.DMA,
            pltpu.SemaphoreType.DMA,
            pltpu.SemaphoreType.DMA,
        ],
    )
    return kern(h, fmi, atomic_seg, view_seg, e_arr, bnd)


def kernel(x_3d, x_mod, x_proj, feature_map_indexing, atomic_seg, view_seg,
           W_conv, b_conv, w_score, W_fuse, b_fuse):
    N, D = x_3d.shape
    A = x_proj.shape[0]

    DP = x_proj.shape[1]
    wc_bf = W_conv.astype(jnp.bfloat16)
    w1_bf = W_fuse[:D].astype(jnp.bfloat16)
    w2_bf = W_fuse[D:].astype(jnp.bfloat16)
    wpad = jnp.zeros((DP, 128), jnp.float32).at[:, 0].set(w_score)
    wpad_bf = wpad.astype(jnp.bfloat16)

    h = _conv_matmul(x_mod, wc_bf, b_conv.reshape(1, D))
    e2d = _score_kernel(x_proj, wpad_bf)
    e_arr = e2d.reshape(A)
    part = _part_matmul(x_3d, w1_bf, b_fuse.reshape(1, D))

    PW = N // NW
    p_starts = (jnp.arange(NW + 1, dtype=jnp.int32) * PW)
    a_starts = jnp.searchsorted(view_seg, p_starts).astype(jnp.int32)
    m_starts = jnp.searchsorted(atomic_seg, a_starts).astype(jnp.int32)
    bnd = jnp.zeros((128,), jnp.int32)
    bnd = bnd.at[0:NW + 1].set(a_starts).at[64:64 + NW + 1].set(m_starts)

    v, counts = _sc_pass(h, feature_map_indexing.astype(jnp.int32),
                         atomic_seg.astype(jnp.int32),
                         view_seg.astype(jnp.int32), e_arr, bnd, N, PW)

    fused = _fuse_matmul(part, v, w2_bf)
    return fused, counts > 0


# speedup vs baseline: 1.0277x; 1.0007x over previous
import dataclasses
import functools

import jax
import jax.numpy as jnp
from jax import lax
from jax.experimental import pallas as pl
from jax.experimental.pallas import tpu as pltpu
from jax.experimental.pallas import tpu_sc as plsc

NC, NS, L = 2, 16, 16
NW = NC * NS
CM = 64
CVW = 128


def _conv_body(x_ref, w_ref, b_ref, o_ref):
    x = x_ref[...].astype(jnp.bfloat16)
    acc = jnp.dot(x, w_ref[...], preferred_element_type=jnp.float32)
    o_ref[...] = jnp.maximum(acc + b_ref[...], 0.0)


def _conv_matmul(x_mod, w, b2d):
    P, D = x_mod.shape
    BM = 512
    return pl.pallas_call(
        _conv_body,
        grid=(P // BM,),
        in_specs=[
            pl.BlockSpec((BM, D), lambda i: (i, 0)),
            pl.BlockSpec((D, D), lambda i: (0, 0)),
            pl.BlockSpec((1, D), lambda i: (0, 0)),
        ],
        out_specs=pl.BlockSpec((BM, D), lambda i: (i, 0)),
        out_shape=jax.ShapeDtypeStruct((P, D), jnp.float32),
    )(x_mod, w, b2d)


def _score_body(xp_ref, w_ref, e_ref):
    x = xp_ref[...].astype(jnp.bfloat16)
    s = jnp.dot(x, w_ref[...], preferred_element_type=jnp.float32)
    e_ref[...] = jnp.exp(s[:, :1])


def _score_kernel(x_proj, wpad):
    A, DP = x_proj.shape
    BA = 4096
    return pl.pallas_call(
        _score_body,
        grid=(A // BA,),
        in_specs=[
            pl.BlockSpec((BA, DP), lambda i: (i, 0)),
            pl.BlockSpec((DP, 128), lambda i: (0, 0)),
        ],
        out_specs=pl.BlockSpec((BA, 1), lambda i: (i, 0)),
        out_shape=jax.ShapeDtypeStruct((A, 1), jnp.float32),
    )(x_proj, wpad)


def _part_body(x_ref, w_ref, b_ref, o_ref):
    x = x_ref[...].astype(jnp.bfloat16)
    acc = jnp.dot(x, w_ref[...], preferred_element_type=jnp.float32)
    o_ref[...] = acc + b_ref[...]


def _part_matmul(x_3d, w1, b2d):
    N, D = x_3d.shape
    BM = 512
    return pl.pallas_call(
        _part_body,
        grid=(N // BM,),
        in_specs=[
            pl.BlockSpec((BM, D), lambda i: (i, 0)),
            pl.BlockSpec((D, D), lambda i: (0, 0)),
            pl.BlockSpec((1, D), lambda i: (0, 0)),
        ],
        out_specs=pl.BlockSpec((BM, D), lambda i: (i, 0)),
        out_shape=jax.ShapeDtypeStruct((N, D), jnp.float32),
    )(x_3d, w1, b2d)


def _fuse_body(p_ref, v_ref, w2_ref, o_ref):
    v = v_ref[...].astype(jnp.bfloat16)
    acc = p_ref[...] + jnp.dot(v, w2_ref[...],
                               preferred_element_type=jnp.float32)
    o_ref[...] = jnp.maximum(acc, 0.0)


def _fuse_matmul(part, v, w2):
    N, D = part.shape
    BM = 512
    return pl.pallas_call(
        _fuse_body,
        grid=(N // BM,),
        in_specs=[
            pl.BlockSpec((BM, D), lambda i: (i, 0)),
            pl.BlockSpec((BM, D), lambda i: (i, 0)),
            pl.BlockSpec((D, D), lambda i: (0, 0)),
        ],
        out_specs=pl.BlockSpec((BM, D), lambda i: (i, 0)),
        out_shape=jax.ShapeDtypeStruct((N, D), jnp.float32),
    )(part, v, w2)


def _sload(ref, idx):
    return ref[pl.ds(idx, L)][0]


def _sc_body(PW, D, h_hbm, fmi_hbm, aseg_hbm, vseg_hbm, e_hbm,
             bnd_hbm, v_hbm, cnt_hbm,
             bnd_v, idxa_v, idxb_v, asega_v, asegb_v, ga_v, gb_v,
             vs_win, e_win, denom_v, cnt_v, pacc_v, stage_v, zero_v,
             sema, semb, semz):
    ND = D // L
    wid = lax.axis_index("s") * NC + lax.axis_index("c")
    pltpu.sync_copy(bnd_hbm, bnd_v)
    a0 = _sload(bnd_v, wid)
    a1 = _sload(bnd_v, wid + 1)
    m0 = _sload(bnd_v, 64 + wid)
    m1 = _sload(bnd_v, 64 + wid + 1)
    p0 = wid * PW

    zeros = jnp.zeros((L,), jnp.float32)

    @pl.loop(0, PW // L)
    def _(i):
        denom_v[pl.ds(i * L, L)] = zeros
        cnt_v[pl.ds(i * L, L)] = zeros

    @pl.loop(0, 16)
    def _(i):
        @pl.loop(0, ND)
        def _(d):
            zero_v[i, pl.ds(d * L, L)] = zeros
            stage_v[i, pl.ds(d * L, L)] = zeros

    @pl.loop(0, ND)
    def _(i):
        pacc_v[pl.ds(i * L, L)] = zeros

    @pl.loop(0, PW // 16)
    def _(i):
        off = pl.multiple_of(p0 + i * 16, 16)
        pltpu.async_copy(zero_v, v_hbm.at[pl.ds(off, 16)], semz)

    iota = lax.iota(jnp.int32, L)
    ones = jnp.ones((L,), jnp.float32)
    c0 = a0 // CVW
    c1 = jnp.maximum(c0, (a1 + CVW - 1) // CVW)

    def devchunk(c, _):
        base = pl.multiple_of(c * CVW, CVW)
        pltpu.sync_copy(vseg_hbm.at[pl.ds(base, CVW)],
                        vs_win.at[pl.ds(0, CVW)])
        pltpu.sync_copy(e_hbm.at[pl.ds(base, CVW)], e_win.at[pl.ds(0, CVW)])
        for k in range(CVW // L):
            g = base + k * L + iota
            act = (g >= a0) & (g < a1)
            idx = vs_win[pl.ds(k * L, L)] - p0
            idx = jnp.where(act, idx, 0)
            plsc.addupdate_scatter(denom_v, [idx], e_win[pl.ds(k * L, L)],
                                   mask=act)
            plsc.addupdate_scatter(cnt_v, [idx], ones, mask=act)
        return 0

    lax.fori_loop(c0, c1, devchunk, 0)
    pltpu.sync_copy(cnt_v, cnt_hbm.at[pl.ds(pl.multiple_of(p0, PW), PW)])

    @pl.loop(0, PW // 16)
    def _(i):
        off = pl.multiple_of(p0 + i * 16, 16)
        pltpu.make_async_copy(zero_v, v_hbm.at[pl.ds(off, 16)], semz).wait()

    def flush(blk):
        pltpu.sync_copy(stage_v, v_hbm.at[pl.ds(pl.multiple_of(blk, 16), 16)])
        @pl.loop(0, 16)
        def _(i):
            @pl.loop(0, ND)
            def _(d):
                stage_v[i, pl.ds(d * L, L)] = zeros

    def close_point(cur_p, blk):
        srow = cur_p - blk
        for d in range(ND):
            sl = pl.ds(d * L, L)
            stage_v[srow, sl] = pacc_v[sl]
            pacc_v[sl] = zeros

    mc0 = m0 // CM
    mc1 = jnp.maximum(mc0, (m1 + CM - 1) // CM)
    nchunks = mc1 - mc0

    def issue(ci, idx_ref, aseg_ref, g_ref, sem):
        base = pl.multiple_of(ci * CM, CM)
        pltpu.sync_copy(fmi_hbm.at[pl.ds(base, CM)], idx_ref)
        pltpu.sync_copy(aseg_hbm.at[pl.ds(base, CM)],
                        aseg_ref.at[pl.ds(0, CM)])
        pltpu.async_copy(h_hbm.at[idx_ref], g_ref, sem)

    def wait_gather(idx_ref, g_ref, sem):
        pltpu.make_async_copy(h_hbm.at[idx_ref], g_ref, sem).wait()

    def walk_chunk(ci, g_ref, aseg_ref, carry):
        base = ci * CM
        r_lo = jnp.maximum(m0 - base, 0)
        r_hi = jnp.maximum(r_lo, jnp.minimum(CM, m1 - base))

        def row_body(r, carry):
            scal, acc = carry
            cur_view = scal[0]
            aseg = _sload(aseg_ref, r)

            def same(carry):
                scal, acc = carry
                nacc = tuple(
                    jnp.maximum(acc[d], g_ref[r, pl.ds(d * L, L)])
                    for d in range(ND))
                return (scal, nacc)

            def diff(carry):
                scal, acc = carry
                cur_view, cur_e, cur_p, blk, wb = scal

                @pl.when(cur_view >= 0)
                def _():
                    for d in range(ND):
                        sl = pl.ds(d * L, L)
                        pacc_v[sl] = pacc_v[sl] + cur_e * acc[d]

                def refill(w):
                    nwb = pl.multiple_of((aseg // CVW) * CVW, CVW)
                    pltpu.sync_copy(e_hbm.at[pl.ds(nwb, CVW)],
                                    e_win.at[pl.ds(0, CVW)])
                    pltpu.sync_copy(vseg_hbm.at[pl.ds(nwb, CVW)],
                                    vs_win.at[pl.ds(0, CVW)])
                    return nwb

                wb = lax.cond(aseg - wb >= CVW, refill, lambda w: w, wb)
                off = aseg - wb
                pnew = _sload(vs_win, off)
                wvec = e_win[pl.ds(off, L)] / denom_v[pl.ds(pnew - p0, L)]
                enew = wvec[0]

                def new_point(blk):
                    @pl.when(cur_p >= 0)
                    def _():
                        close_point(cur_p, blk)
                    nblk = (pnew // 16) * 16

                    @pl.when((blk >= 0) & (nblk != blk))
                    def _():
                        flush(blk)
                    return nblk

                blk = lax.cond(pnew != cur_p, new_point, lambda b: b, blk)
                cur_p = jnp.where(pnew != cur_p, pnew, cur_p)

                nacc = tuple(g_ref[r, pl.ds(d * L, L)] for d in range(ND))
                return ((aseg, enew, cur_p, blk, wb), nacc)

            return lax.cond(aseg == cur_view, same, diff, carry)

        return lax.fori_loop(r_lo, r_hi, row_body, carry)

    @pl.when(nchunks > 0)
    def _():
        issue(mc0, idxa_v, asega_v, ga_v, sema)

    init = ((jnp.int32(-1), jnp.float32(0.0), jnp.int32(-1), jnp.int32(-1),
             jnp.int32(-2 * CVW)),
            tuple(zeros for _ in range(ND)))
    npairs = (nchunks + 1) // 2

    def pair_body(k, carry):
        ci0 = mc0 + 2 * k
        ci1 = ci0 + 1

        @pl.when(ci1 < mc1)
        def _():
            issue(ci1, idxb_v, asegb_v, gb_v, semb)

        wait_gather(idxa_v, ga_v, sema)
        carry = walk_chunk(ci0, ga_v, asega_v, carry)

        @pl.when(ci0 + 2 < mc1)
        def _():
            issue(ci0 + 2, idxa_v, asega_v, ga_v, sema)

        def do_b(carry):
            wait_gather(idxb_v, gb_v, semb)
            return walk_chunk(ci1, gb_v, asegb_v, carry)

        return lax.cond(ci1 < mc1, do_b, lambda c: c, carry)

    scal, acc = lax.fori_loop(0, npairs, pair_body, init)
    cur_view, cur_e, cur_p, blk, _ = scal

    @pl.when(cur_view >= 0)
    def _():
        for d in range(ND):
            sl = pl.ds(d * L, L)
            pacc_v[sl] = pacc_v[sl] + cur_e * acc[d]

    @pl.when(cur_p >= 0)
    def _():
        close_point(cur_p, blk)
        flush(blk)


def _sc_pass(h, fmi, atomic_seg, view_seg, e_arr, bnd, N, PW):
    D = h.shape[1]
    mesh = plsc.VectorSubcoreMesh(core_axis_name="c", subcore_axis_name="s")
    f32 = jnp.float32
    cp = pltpu.CompilerParams()
    if "needs_layout_passes" in pltpu.CompilerParams.__dataclass_fields__:
        cp = dataclasses.replace(cp, needs_layout_passes=False)
    kern = pl.kernel(
        functools.partial(_sc_body, PW, D),
        out_type=[
            jax.ShapeDtypeStruct((N, D), f32),
            jax.ShapeDtypeStruct((N,), f32),
        ],
        mesh=mesh,
        compiler_params=cp,
        scratch_types=[
            pltpu.VMEM((128,), jnp.int32),
            pltpu.VMEM((CM,), jnp.int32),
            pltpu.VMEM((CM,), jnp.int32),
            pltpu.VMEM((CM + L,), jnp.int32),
            pltpu.VMEM((CM + L,), jnp.int32),
            pltpu.VMEM((CM, D), f32),
            pltpu.VMEM((CM, D), f32),
            pltpu.VMEM((CVW + L,), jnp.int32),
            pltpu.VMEM((CVW + L,), f32),
            pltpu.VMEM((PW + L,), f32),
            pltpu.VMEM((PW,), f32),
            pltpu.VMEM((D,), f32),
            pltpu.VMEM((16, D), f32),
            pltpu.VMEM((16, D), f32),
            pltpu.SemaphoreType.DMA,
            pltpu.SemaphoreType.DMA,
            pltpu.SemaphoreType.DMA,
        ],
    )
    return kern(h, fmi, atomic_seg, view_seg, e_arr, bnd)


def kernel(x_3d, x_mod, x_proj, feature_map_indexing, atomic_seg, view_seg,
           W_conv, b_conv, w_score, W_fuse, b_fuse):
    N, D = x_3d.shape
    A = x_proj.shape[0]

    DP = x_proj.shape[1]
    wc_bf = W_conv.astype(jnp.bfloat16)
    w1_bf = W_fuse[:D].astype(jnp.bfloat16)
    w2_bf = W_fuse[D:].astype(jnp.bfloat16)
    wpad = jnp.zeros((DP, 128), jnp.float32).at[:, 0].set(w_score)
    wpad_bf = wpad.astype(jnp.bfloat16)

    h = _conv_matmul(x_mod, wc_bf, b_conv.reshape(1, D))
    e2d = _score_kernel(x_proj, wpad_bf)
    e_arr = e2d.reshape(A)
    part = _part_matmul(x_3d, w1_bf, b_fuse.reshape(1, D))

    PW = N // NW
    p_starts = (jnp.arange(NW + 1, dtype=jnp.int32) * PW)
    a_starts = jnp.searchsorted(view_seg, p_starts).astype(jnp.int32)
    m_starts = jnp.searchsorted(atomic_seg, a_starts).astype(jnp.int32)
    bnd = jnp.zeros((128,), jnp.int32)
    bnd = bnd.at[0:NW + 1].set(a_starts).at[64:64 + NW + 1].set(m_starts)

    v, counts = _sc_pass(h, feature_map_indexing.astype(jnp.int32),
                         atomic_seg.astype(jnp.int32),
                         view_seg.astype(jnp.int32), e_arr, bnd, N, PW)

    fused = _fuse_matmul(part, v, w2_bf)
    return fused, counts > 0
